# baseline (speedup 1.0000x reference)
import jax
import jax.numpy as jnp
from jax import lax
from jax.experimental import pallas as pl
from jax.experimental.pallas import tpu as pltpu
from jax.experimental.pallas import tpu_sc as plsc

_K = 8192
_D = 256
_HALF = 4096
_NQ = 4
_RB = 2
_TG = _HALF // 128
_QG = _TG // _NQ
_TSTEP = _QG // _RB
_NT = 32
_SCROWS = (_K - _HALF) // _NT


def _tc_dist_body(z_ref, cb0, cb1, cb2, cb3, o0, o1, o2, o3):
    z = z_ref[:].reshape(1, 1, _D)
    for cb, o in ((cb0, o0), (cb1, o1), (cb2, o2), (cb3, o3)):
        e = cb[:] - z
        o[:] = jnp.sum(e * e, axis=-1).reshape(1, _RB, 128)


def _sc_dist_body(z_hbm, cb_hbm, out_hbm, z_v, cb_v, dout_v, sem):
    cid = lax.axis_index("c")
    sid = lax.axis_index("s")
    wid = sid * 2 + cid
    rowbase = wid * _SCROWS
    pltpu.sync_copy(z_hbm, z_v)
    pltpu.sync_copy(cb_hbm.at[pl.ds(rowbase * _D, _SCROWS * _D)], cb_v)
    zc = [z_v[pl.ds(ch * 16, 16)] for ch in range(_D // 16)]
    iota = lax.iota(jnp.int32, 16)

    def group(g, _):
        sums = jnp.zeros((16,), jnp.float32)
        for r in range(16):
            a0 = jnp.zeros((16,), jnp.float32)
            a1 = jnp.zeros((16,), jnp.float32)
            a2 = jnp.zeros((16,), jnp.float32)
            a3 = jnp.zeros((16,), jnp.float32)
            accs = [a0, a1, a2, a3]
            for ch in range(_D // 16):
                c = cb_v[pl.ds((g * 16 + r) * _D + ch * 16, 16)]
                t = c - zc[ch]
                accs[ch % 4] = accs[ch % 4] + t * t
            s = jnp.sum((accs[0] + accs[1]) + (accs[2] + accs[3]))
            sums = jnp.where(iota == r, jnp.full((16,), s, jnp.float32), sums)
        dout_v[pl.ds(g * 16, 16)] = sums
        return 0

    lax.fori_loop(0, _SCROWS // 16, group, 0)
    pltpu.sync_copy(dout_v, out_hbm.at[pl.ds(rowbase, _SCROWS)])


def _sel_body(sym_ref, d0, d1, d2, d3, dsc, adj_ref, out_ref, rows_ref, sem):
    cur = sym_ref[0]
    pltpu.make_async_copy(
        adj_ref.at[pl.ds(cur, 1)], rows_ref.at[pl.ds(0, 1)], sem
    ).start()
    d = jnp.concatenate([d0[:], d1[:], d2[:], d3[:], dsc[:]], axis=0)
    ri = lax.broadcasted_iota(jnp.int32, d.shape, 0)
    ci = lax.broadcasted_iota(jnp.int32, d.shape, 1)
    flat = ri * 128 + ci

    idxs, vals = [], []
    for t in range(5):
        m = jnp.min(d)
        idx = jnp.min(jnp.where(d == m, flat, jnp.int32(1 << 30)))
        pltpu.make_async_copy(
            adj_ref.at[pl.ds(idx, 1)], rows_ref.at[pl.ds(t + 1, 1)], sem
        ).start()
        idxs.append(idx)
        vals.append(m)
        d = jnp.where(flat == idx, jnp.float32(jnp.inf), d)

    for t in range(6):
        pltpu.make_async_copy(
            adj_ref.at[pl.ds(0, 1)], rows_ref.at[pl.ds(t, 1)], sem
        ).wait()

    cur_row = rows_ref[pl.ds(0, 1), :]
    best = jnp.int32(0)
    bs = jnp.float32(0)
    for t in range(5):
        gd = jnp.mean(jnp.abs(cur_row - rows_ref[pl.ds(t + 1, 1), :]))
        sc = -vals[t] + jnp.float32(0.1) * gd
        sc = jnp.where(idxs[t] == cur, -jnp.inf, sc)
        if t == 0:
            best, bs = idxs[t], sc
        else:
            take = sc > bs
            best = jnp.where(take, idxs[t], best)
            bs = jnp.maximum(bs, sc)
    out_ref[0] = best


def kernel(z_flat, codebook, adjacency, current_sym):
    sym = jnp.asarray(current_sym, dtype=jnp.int32).reshape(1)
    z2 = z_flat.reshape(1, _D)
    cb3 = codebook.reshape(_K // 128, 128, _D)
    cbflat = codebook.reshape(_K * _D)

    dsc = pl.kernel(
        _sc_dist_body,
        out_type=jax.ShapeDtypeStruct((_K - _HALF,), jnp.float32),
        mesh=plsc.VectorSubcoreMesh(core_axis_name="c", subcore_axis_name="s",
                                    num_cores=2),
        compiler_params=pltpu.CompilerParams(needs_layout_passes=False),
        scratch_types=[
            pltpu.VMEM((_D,), jnp.float32),
            pltpu.VMEM((_SCROWS * _D,), jnp.float32),
            pltpu.VMEM((_SCROWS,), jnp.float32),
            pltpu.SemaphoreType.DMA,
        ],
    )(z_flat, cbflat)

    def _mk_spec(q):
        return pl.BlockSpec(
            (_RB, 128, _D), lambda i, q=q: (q * _TSTEP + i, 0, 0)
        )

    douts = pl.pallas_call(
        _tc_dist_body,
        grid=(_TSTEP,),
        in_specs=[
            pl.BlockSpec((1, _D), lambda i: (0, 0)),
            _mk_spec(0), _mk_spec(1), _mk_spec(2), _mk_spec(3),
        ],
        out_specs=[
            pl.BlockSpec((1, _RB, 128), lambda i: (i, 0, 0)) for _ in range(_NQ)
        ],
        out_shape=[
            jax.ShapeDtypeStruct((_TSTEP, _RB, 128), jnp.float32)
            for _ in range(_NQ)
        ],
    )(z2, cb3, cb3, cb3, cb3)
    douts = [o.reshape(_QG, 128) for o in douts]

    out = pl.pallas_call(
        _sel_body,
        in_specs=[
            pl.BlockSpec(memory_space=pltpu.SMEM),
            pl.BlockSpec((_QG, 128), lambda: (0, 0)),
            pl.BlockSpec((_QG, 128), lambda: (0, 0)),
            pl.BlockSpec((_QG, 128), lambda: (0, 0)),
            pl.BlockSpec((_QG, 128), lambda: (0, 0)),
            pl.BlockSpec(((_K - _HALF) // 128, 128), lambda: (0, 0)),
            pl.BlockSpec(memory_space=pl.ANY),
        ],
        out_specs=pl.BlockSpec(memory_space=pltpu.SMEM),
        out_shape=jax.ShapeDtypeStruct((1,), jnp.int32),
        scratch_shapes=[
            pltpu.VMEM((8, _K), jnp.float32),
            pltpu.SemaphoreType.DMA,
        ],
    )(sym, douts[0], douts[1], douts[2], douts[3],
      dsc.reshape((_K - _HALF) // 128, 128), adjacency)
    return out[0]


# speedup vs baseline: 1.2824x; 1.2824x over previous
import jax
import jax.numpy as jnp
from jax import lax
from jax.experimental import pallas as pl
from jax.experimental.pallas import tpu as pltpu
from jax.experimental.pallas import tpu_sc as plsc

_K = 8192
_D = 256
_HALF = 5120
_NQ = 4
_RB = 2
_TG = _HALF // 128
_QG = _TG // _NQ
_TSTEP = _QG // _RB
_NT = 32
_SCROWS = (_K - _HALF) // _NT


def _tc_dist_body(z_ref, cb0, cb1, cb2, cb3, o0, o1, o2, o3):
    z = z_ref[:].reshape(1, 1, _D)
    for cb, o in ((cb0, o0), (cb1, o1), (cb2, o2), (cb3, o3)):
        e = cb[:] - z
        o[:] = jnp.sum(e * e, axis=-1).reshape(1, _RB, 128)


def _sc_dist_body(z_hbm, cb_hbm, out_hbm, z_v, cb_v, dout_v, sem):
    cid = lax.axis_index("c")
    sid = lax.axis_index("s")
    wid = sid * 2 + cid
    rowbase = wid * _SCROWS
    pltpu.sync_copy(z_hbm, z_v)
    pltpu.sync_copy(cb_hbm.at[pl.ds(_HALF + rowbase, _SCROWS)], cb_v)
    zc = [z_v[pl.ds(ch * 16, 16)] for ch in range(_D // 16)]
    iota = lax.iota(jnp.int32, 16)

    def group(g, _):
        sums = jnp.zeros((16,), jnp.float32)
        for r in range(16):
            a0 = jnp.zeros((16,), jnp.float32)
            a1 = jnp.zeros((16,), jnp.float32)
            a2 = jnp.zeros((16,), jnp.float32)
            a3 = jnp.zeros((16,), jnp.float32)
            accs = [a0, a1, a2, a3]
            for ch in range(_D // 16):
                c = cb_v[g * 16 + r, pl.ds(ch * 16, 16)]
                t = c - zc[ch]
                accs[ch % 4] = accs[ch % 4] + t * t
            s = jnp.sum((accs[0] + accs[1]) + (accs[2] + accs[3]))
            sums = jnp.where(iota == r, jnp.full((16,), s, jnp.float32), sums)
        dout_v[pl.ds(g * 16, 16)] = sums
        return 0

    lax.fori_loop(0, _SCROWS // 16, group, 0)
    pltpu.sync_copy(dout_v, out_hbm.at[pl.ds(rowbase, _SCROWS)])


def _sel_body(sym_ref, d0, d1, d2, d3, dsc, adj_ref, out_ref, rows_ref, sem):
    cur = sym_ref[0]
    pltpu.make_async_copy(
        adj_ref.at[pl.ds(cur, 1)], rows_ref.at[pl.ds(0, 1)], sem
    ).start()
    d = jnp.concatenate([d0[:], d1[:], d2[:], d3[:], dsc[:]], axis=0)
    ri = lax.broadcasted_iota(jnp.int32, d.shape, 0)
    ci = lax.broadcasted_iota(jnp.int32, d.shape, 1)
    flat = ri * 128 + ci

    idxs, vals = [], []
    for t in range(5):
        m = jnp.min(d)
        idx = jnp.min(jnp.where(d == m, flat, jnp.int32(1 << 30)))
        pltpu.make_async_copy(
            adj_ref.at[pl.ds(idx, 1)], rows_ref.at[pl.ds(t + 1, 1)], sem
        ).start()
        idxs.append(idx)
        vals.append(m)
        d = jnp.where(flat == idx, jnp.float32(jnp.inf), d)

    for t in range(6):
        pltpu.make_async_copy(
            adj_ref.at[pl.ds(0, 1)], rows_ref.at[pl.ds(t, 1)], sem
        ).wait()

    cur_row = rows_ref[pl.ds(0, 1), :]
    best = jnp.int32(0)
    bs = jnp.float32(0)
    for t in range(5):
        gd = jnp.mean(jnp.abs(cur_row - rows_ref[pl.ds(t + 1, 1), :]))
        sc = -vals[t] + jnp.float32(0.1) * gd
        sc = jnp.where(idxs[t] == cur, -jnp.inf, sc)
        if t == 0:
            best, bs = idxs[t], sc
        else:
            take = sc > bs
            best = jnp.where(take, idxs[t], best)
            bs = jnp.maximum(bs, sc)
    out_ref[0] = best


def kernel(z_flat, codebook, adjacency, current_sym):
    sym = jnp.asarray(current_sym, dtype=jnp.int32).reshape(1)
    z2 = z_flat.reshape(1, _D)
    cb3 = codebook.reshape(_K // 128, 128, _D)

    dsc = pl.kernel(
        _sc_dist_body,
        out_type=jax.ShapeDtypeStruct((_K - _HALF,), jnp.float32),
        mesh=plsc.VectorSubcoreMesh(core_axis_name="c", subcore_axis_name="s",
                                    num_cores=2),
        compiler_params=pltpu.CompilerParams(needs_layout_passes=False),
        scratch_types=[
            pltpu.VMEM((_D,), jnp.float32),
            pltpu.VMEM((_SCROWS, _D), jnp.float32),
            pltpu.VMEM((_SCROWS,), jnp.float32),
            pltpu.SemaphoreType.DMA,
        ],
    )(z_flat, codebook)

    def _mk_spec(q):
        return pl.BlockSpec(
            (_RB, 128, _D), lambda i, q=q: (q * _TSTEP + i, 0, 0)
        )

    douts = pl.pallas_call(
        _tc_dist_body,
        grid=(_TSTEP,),
        in_specs=[
            pl.BlockSpec((1, _D), lambda i: (0, 0)),
            _mk_spec(0), _mk_spec(1), _mk_spec(2), _mk_spec(3),
        ],
        out_specs=[
            pl.BlockSpec((1, _RB, 128), lambda i: (i, 0, 0)) for _ in range(_NQ)
        ],
        out_shape=[
            jax.ShapeDtypeStruct((_TSTEP, _RB, 128), jnp.float32)
            for _ in range(_NQ)
        ],
    )(z2, cb3, cb3, cb3, cb3)
    douts = [o.reshape(_QG, 128) for o in douts]

    out = pl.pallas_call(
        _sel_body,
        in_specs=[
            pl.BlockSpec(memory_space=pltpu.SMEM),
            pl.BlockSpec((_QG, 128), lambda: (0, 0)),
            pl.BlockSpec((_QG, 128), lambda: (0, 0)),
            pl.BlockSpec((_QG, 128), lambda: (0, 0)),
            pl.BlockSpec((_QG, 128), lambda: (0, 0)),
            pl.BlockSpec(((_K - _HALF) // 128, 128), lambda: (0, 0)),
            pl.BlockSpec(memory_space=pl.ANY),
        ],
        out_specs=pl.BlockSpec(memory_space=pltpu.SMEM),
        out_shape=jax.ShapeDtypeStruct((1,), jnp.int32),
        scratch_shapes=[
            pltpu.VMEM((8, _K), jnp.float32),
            pltpu.SemaphoreType.DMA,
        ],
    )(sym, douts[0], douts[1], douts[2], douts[3],
      dsc.reshape((_K - _HALF) // 128, 128), adjacency)
    return out[0]


# speedup vs baseline: 3.7342x; 2.9120x over previous
import jax
import jax.numpy as jnp
from jax import lax
from jax.experimental import pallas as pl
from jax.experimental.pallas import tpu as pltpu

_K = 8192
_D = 256
_RB = 8
_NSTEP = _K // 128 // _RB


def _body(sym_ref, z_ref, cb_ref, adj_ref, out_ref, dists_ref, rows_ref, sem):
    i = pl.program_id(0)

    @pl.when(i == 0)
    def _start_cur_row():
        pltpu.make_async_copy(
            adj_ref.at[pl.ds(sym_ref[0], 1)], rows_ref.at[pl.ds(0, 1)], sem
        ).start()

    @pl.when(i < _NSTEP)
    def _dist_step():
        z = z_ref[:].reshape(1, 1, _D)
        e = cb_ref[:] - z
        d = jnp.sum(e * e, axis=-1)
        dists_ref[pl.ds(i * _RB, _RB), :] = d

    @pl.when(i == _NSTEP)
    def _select():
        d = dists_ref[:]
        ri = lax.broadcasted_iota(jnp.int32, d.shape, 0)
        ci = lax.broadcasted_iota(jnp.int32, d.shape, 1)
        flat = ri * 128 + ci
        cur = sym_ref[0]

        idxs, vals = [], []
        for t in range(5):
            m = jnp.min(d)
            idx = jnp.min(jnp.where(d == m, flat, jnp.int32(1 << 30)))
            pltpu.make_async_copy(
                adj_ref.at[pl.ds(idx, 1)], rows_ref.at[pl.ds(t + 1, 1)], sem
            ).start()
            idxs.append(idx)
            vals.append(m)
            d = jnp.where(flat == idx, jnp.float32(jnp.inf), d)

        for t in range(6):
            pltpu.make_async_copy(
                adj_ref.at[pl.ds(0, 1)], rows_ref.at[pl.ds(t, 1)], sem
            ).wait()

        cur_row = rows_ref[pl.ds(0, 1), :]
        best = jnp.int32(0)
        bs = jnp.float32(0)
        for t in range(5):
            gd = jnp.mean(jnp.abs(cur_row - rows_ref[pl.ds(t + 1, 1), :]))
            sc = -vals[t] + jnp.float32(0.1) * gd
            sc = jnp.where(idxs[t] == cur, -jnp.inf, sc)
            if t == 0:
                best, bs = idxs[t], sc
            else:
                take = sc > bs
                best = jnp.where(take, idxs[t], best)
                bs = jnp.maximum(bs, sc)
        out_ref[0] = best


def kernel(z_flat, codebook, adjacency, current_sym):
    sym = jnp.asarray(current_sym, dtype=jnp.int32).reshape(1)
    z2 = z_flat.reshape(1, _D)
    cb3 = codebook.reshape(_K // 128, 128, _D)
    out = pl.pallas_call(
        _body,
        grid=(_NSTEP + 1,),
        in_specs=[
            pl.BlockSpec(memory_space=pltpu.SMEM),
            pl.BlockSpec((1, _D), lambda i: (0, 0)),
            pl.BlockSpec((_RB, 128, _D), lambda i: (jnp.minimum(i, _NSTEP - 1), 0, 0)),
            pl.BlockSpec(memory_space=pl.ANY),
        ],
        out_specs=pl.BlockSpec(memory_space=pltpu.SMEM),
        out_shape=jax.ShapeDtypeStruct((1,), jnp.int32),
        scratch_shapes=[
            pltpu.VMEM((_K // 128, 128), jnp.float32),
            pltpu.VMEM((8, _K), jnp.float32),
            pltpu.SemaphoreType.DMA,
        ],
    )(sym, z2, cb3, adjacency)
    return out[0]
